# baseline (speedup 1.0000x reference)
import jax
import jax.numpy as jnp
from jax.experimental import pallas as pl

N = 10000
E = 320000
D = 128
B = 256


def _gcn_layer(x, src, dst, ew, W, b):
    n = x.shape[0]
    loop = jnp.arange(n, dtype=src.dtype)
    s = jnp.concatenate([src, loop])
    d = jnp.concatenate([dst, loop])
    w = jnp.concatenate([ew, jnp.ones((n,), x.dtype)])
    deg = jax.ops.segment_sum(w, d, num_segments=n)
    dinv = jnp.where(deg > 0, jax.lax.rsqrt(jnp.maximum(deg, 1e-12)), 0.0)
    norm = dinv[s] * w * dinv[d]
    h = x @ W
    msg = h[s] * norm[:, None]
    out = jax.ops.segment_sum(msg, d, num_segments=n) + b
    return out


def _gnn_pool(x, edge_index, ew, batch, W1, b1, W2, b2, W3, b3):
    src, dst = edge_index[0], edge_index[1]
    h = jnp.maximum(_gcn_layer(x, src, dst, ew, W1, b1), 0.0)
    h = jnp.maximum(_gcn_layer(h, src, dst, ew, W2, b2), 0.0)
    h = _gcn_layer(h, src, dst, ew, W3, b3)
    sums = jax.ops.segment_sum(h, batch, num_segments=B)
    cnts = jax.ops.segment_sum(jnp.ones((h.shape[0],), h.dtype), batch, num_segments=B)
    return sums / jnp.maximum(cnts, 1.0)[:, None]


def _mlp_body(mp_ref, mn_ref, p1a_ref, p1b_ref, b1_ref, p2_ref, b2_ref, o_ref):
    hp = jax.lax.dot(mp_ref[...], p1a_ref[...],
                     precision=jax.lax.Precision.HIGHEST,
                     preferred_element_type=jnp.float32)
    hn = jax.lax.dot(mn_ref[...], p1b_ref[...],
                     precision=jax.lax.Precision.HIGHEST,
                     preferred_element_type=jnp.float32)
    z = jnp.maximum(hp + hn + b1_ref[...], 0.0)
    o_ref[...] = jax.lax.dot(z, p2_ref[...],
                             precision=jax.lax.Precision.HIGHEST,
                             preferred_element_type=jnp.float32) + b2_ref[...]


def kernel(x_pos, edge_index_pos, edge_attr_pos, batch_pos, x_neg, edge_index_neg, edge_attr_neg, batch_neg, W1, b1, W2, b2, W3, b3, P1W, P1b, P2W, P2b):
    mp = _gnn_pool(x_pos, edge_index_pos, edge_attr_pos, batch_pos, W1, b1, W2, b2, W3, b3)
    mn = _gnn_pool(x_neg, edge_index_neg, edge_attr_neg, batch_neg, W1, b1, W2, b2, W3, b3)
    p2 = jnp.zeros((D, D), jnp.float32).at[:, :2].set(P2W)
    b2 = jnp.zeros((1, D), jnp.float32).at[:, :2].set(P2b)
    out = pl.pallas_call(
        _mlp_body,
        out_shape=jax.ShapeDtypeStruct((B, D), jnp.float32),
    )(mp, mn, P1W[:D], P1W[D:], P1b.reshape(1, D), p2, b2)
    return out[:, :2]


# speedup vs baseline: 6.6856x; 6.6856x over previous
import functools

import jax
import jax.numpy as jnp
from jax import lax
from jax.experimental import pallas as pl
from jax.experimental.pallas import tpu as pltpu
from jax.experimental.pallas import tpu_sc as plsc

N = 10000
E = 320000
D = 128
B = 256
NP = 10240
BP = 264

NC = 2
NS = 16
CH = 128
NU = 160
DGRP = 4
DNG = 80
EPAD = NS * NU * CH
RPT = NP // NS

_F32 = jnp.float32
_SC_PARAMS = pltpu.CompilerParams(needs_layout_passes=False)


@functools.cache
def _mesh():
    return plsc.VectorSubcoreMesh(core_axis_name="c", subcore_axis_name="s",
                                  num_cores=NC, num_subcores=NS)


def _zero_rows(buf, nrows, ncols=D):
    @pl.loop(0, nrows)
    def _(r):
        for j in range(ncols // 16):
            buf[r, pl.ds(j * 16, 16)] = jnp.zeros((16,), _F32)


def _zero_acc_slice(zsrc, acc, s):
    nfull = RPT // CH
    for k in range(nfull):
        pltpu.sync_copy(zsrc, acc.at[pl.ds(s * RPT + k * CH, CH)])
    rem = RPT - nfull * CH
    if rem:
        pltpu.sync_copy(zsrc.at[pl.ds(0, rem)],
                        acc.at[pl.ds(s * RPT + nfull * CH, rem)])


def _sc_deg_body(eb_hbm, ew_hbm, deg_hbm, eb_v, w_v, stage, zc, acc,
                 dsem, esem):
    c = lax.axis_index("c")
    s = lax.axis_index("s")
    eb_g = eb_hbm.at[c].at[s]
    ew_g = ew_hbm.at[c].at[s]
    @pl.loop(0, 64)
    def _(r):
        zc[r, :] = jnp.zeros((16,), _F32)
    for k in range(RPT // 64):
        pltpu.sync_copy(zc, acc.at[pl.ds(s * RPT + k * 64, 64)])
    plsc.subcore_barrier()

    def issue_edges(gi, pb):
        pltpu.async_copy(eb_g.at[gi], eb_v.at[pb], esem.at[pb])
        pltpu.async_copy(ew_g.at[gi], w_v.at[pb], esem.at[pb])

    def wait_edges(pb):
        pltpu.make_async_copy(eb_g.at[0], eb_v.at[pb], esem.at[pb]).wait()
        pltpu.make_async_copy(ew_g.at[0], w_v.at[pb], esem.at[pb]).wait()

    def dvec(pb, k, g):
        off = 256 + k * 64 + g * 16
        return eb_v[pb, off // 128, pl.ds(off % 128, 16)]

    def do_group(gi, pb):
        wait_edges(pb)
        @pl.when(gi + 1 < DNG)
        def _():
            issue_edges(gi + 1, 1 - pb)
        for k in range(DGRP):
            b2 = k % 2
            if k < 2:
                @pl.when(gi > 0)
                def _():
                    for g in range(64 // 16):
                        pltpu.make_async_copy(
                            stage.at[b2].at[pl.ds(g * 16, 16)],
                            acc.at[dvec(1 - pb, k + 2, g)],
                            dsem.at[b2]).wait()
            else:
                for g in range(64 // 16):
                    pltpu.make_async_copy(stage.at[b2].at[pl.ds(g * 16, 16)],
                                          acc.at[dvec(pb, k - 2, g)],
                                          dsem.at[b2]).wait()
            for g in range(64 // 16):
                wrow = w_v[pb, (k * 64 + g * 16) // 128,
                           pl.ds((k * 64 + g * 16) % 128, 16)]
                for t in range(16):
                    stage[b2, g * 16 + t, :] = jnp.full((16,), wrow[t])
            for g in range(64 // 16):
                pltpu.async_copy(stage.at[b2].at[pl.ds(g * 16, 16)],
                                 acc.at[dvec(pb, k, g)], dsem.at[b2],
                                 add=True)

    issue_edges(0, 0)
    @pl.loop(0, DNG, step=2)
    def _(gi):
        do_group(gi, 0)
        do_group(gi + 1, 1)

    for b2 in range(2):
        for g in range(64 // 16):
            pltpu.make_async_copy(stage.at[b2].at[pl.ds(g * 16, 16)],
                                  acc.at[dvec(1, 2 + b2, g)],
                                  dsem.at[b2]).wait()

    plsc.subcore_barrier()
    pltpu.sync_copy(acc.at[pl.ds(s * RPT, RPT)],
                    deg_hbm.at[c].at[pl.ds(s * RPT, RPT)])


@functools.cache
def _sc_deg_kernel():
    return pl.kernel(
        _sc_deg_body,
        compiler_params=_SC_PARAMS,
        out_type=jax.ShapeDtypeStruct((NC, NP, 16), _F32),
        mesh=_mesh(),
        scratch_types=[
            pltpu.VMEM((2, 8, 128), jnp.int32),
            pltpu.VMEM((2, 8, 128), _F32),
            pltpu.VMEM((2, 64, 16), _F32),
            pltpu.VMEM((64, 16), _F32),
            pltpu.VMEM_SHARED((NP, 16), _F32),
            pltpu.SemaphoreType.DMA((2,)),
            pltpu.SemaphoreType.DMA((2,)),
        ],
    )


def _sc_deg(eidx, ew):
    return _sc_deg_kernel()(eidx, ew)


def _edge_blocks4(edge_index, ew):
    pad = EPAD - E
    src = jnp.concatenate([edge_index[0], jnp.zeros((pad,), jnp.int32)])
    dst = jnp.concatenate([edge_index[1], jnp.zeros((pad,), jnp.int32)])
    w = jnp.concatenate([ew, jnp.zeros((pad,), _F32)])
    src = src.reshape(NS, DNG, 2, 128)
    dst = dst.reshape(NS, DNG, 2, 128)
    w = w.reshape(NS, DNG, 2, 128)
    zpad = jnp.zeros((NS, DNG, 4, 128), jnp.int32)
    idx = jnp.concatenate([src, dst, zpad], axis=2)
    wblk = jnp.concatenate([w, zpad.astype(_F32),
                            zpad.astype(_F32)[:, :, :2]], axis=2)
    return idx, wblk


def _sc_scatter_body(hp_hbm, eb_hbm, ew_hbm, out_hbm, eb_v, w_v, gbuf, acc,
                     gsem, ssem, esem):
    c = lax.axis_index("c")
    s = lax.axis_index("s")
    hp_g = hp_hbm.at[c]
    eb_g = eb_hbm.at[c].at[s]
    ew_g = ew_hbm.at[c].at[s]

    _zero_rows(gbuf.at[0], CH)
    for k in range(RPT // CH):
        pltpu.sync_copy(gbuf.at[0], acc.at[pl.ds(s * RPT + k * CH, CH)])
    plsc.subcore_barrier()

    def issue_edges(u, b):
        pltpu.async_copy(eb_g.at[u], eb_v.at[b], esem.at[b])
        pltpu.async_copy(ew_g.at[u], w_v.at[b], esem.at[b])

    def wait_edges(b):
        pltpu.make_async_copy(eb_g.at[0], eb_v.at[b], esem.at[b]).wait()
        pltpu.make_async_copy(ew_g.at[0], w_v.at[b], esem.at[b]).wait()

    def issue_gather(b):
        pltpu.async_copy(hp_g.at[eb_v.at[b, 0]], gbuf.at[b], gsem.at[b])

    def wait_gather(b):
        pltpu.make_async_copy(hp_g.at[eb_v.at[b, 0]], gbuf.at[b],
                              gsem.at[b]).wait()

    def issue_scatter(b):
        pltpu.async_copy(gbuf.at[b], acc.at[eb_v.at[b, 1]], ssem.at[b],
                         add=True)

    def wait_scatter(b):
        pltpu.make_async_copy(gbuf.at[b], acc.at[eb_v.at[b, 1]],
                              ssem.at[b]).wait()

    def scale(b):
        for g in range(CH // 16):
            wrow = w_v[b, 0, pl.ds(g * 16, 16)]
            for t in range(16):
                e = g * 16 + t
                wv = jnp.full((16,), wrow[t])
                for j in range(D // 16):
                    sl = pl.ds(j * 16, 16)
                    gbuf[b, e, sl] = gbuf[b, e, sl] * wv

    issue_edges(0, 0)
    wait_edges(0)
    issue_gather(0)

    @pl.loop(0, NU, step=2)
    def _(u):
        for b in range(2):
            ub = u + b
            ob = 1 - b
            wait_gather(b)
            @pl.when(ub > 0)
            def _():
                wait_scatter(ob)
            @pl.when(ub + 1 < NU)
            def _():
                issue_edges(ub + 1, ob)
                wait_edges(ob)
                issue_gather(ob)
            scale(b)
            issue_scatter(b)

    wait_scatter(1)
    plsc.subcore_barrier()

    for k in range(RPT // CH):
        row = s * RPT + k * CH
        pltpu.sync_copy(acc.at[pl.ds(row, CH)],
                        out_hbm.at[c].at[pl.ds(row, CH)])


@functools.cache
def _sc_scatter_kernel():
    return pl.kernel(
        _sc_scatter_body,
        compiler_params=_SC_PARAMS,
        out_type=jax.ShapeDtypeStruct((NC, NP, D), _F32),
        mesh=_mesh(),
        scratch_types=[
            pltpu.VMEM((2, 8, 128), jnp.int32),
            pltpu.VMEM((2, 8, 128), _F32),
            pltpu.VMEM((2, CH, D), _F32),
            pltpu.VMEM_SHARED((NP, D), _F32),
            pltpu.SemaphoreType.DMA((2,)),
            pltpu.SemaphoreType.DMA((2,)),
            pltpu.SemaphoreType.DMA((2,)),
        ],
    )


def _sc_scatter(hp, eidx, ew):
    return _sc_scatter_kernel()(hp, eidx, ew)


def _sc_pool_body(z_hbm, batch_hbm, sums_hbm, cnts_hbm,
                  b_v, zbuf, ones_v, zc, pacc, cacc, psem):
    c = lax.axis_index("c")
    s = lax.axis_index("s")
    rows_b = B // NS
    pltpu.sync_copy(batch_hbm.at[c].at[s], b_v)
    @pl.loop(0, 16)
    def _(r):
        ones_v[r, :] = jnp.full((16,), 1.0)
    @pl.loop(0, rows_b)
    def _(r):
        zc[r, :] = jnp.zeros((16,), _F32)
    _zero_rows(zbuf, rows_b)
    pltpu.sync_copy(zbuf.at[pl.ds(0, rows_b)],
                    pacc.at[pl.ds(s * rows_b, rows_b)])
    pltpu.sync_copy(zc, cacc.at[pl.ds(s * rows_b, rows_b)])
    plsc.subcore_barrier()

    def bvec(k, g):
        off = k * 128 + g * 16
        return b_v[off // 128, pl.ds(off % 128, 16)]

    for k in range(RPT // 128):
        pltpu.sync_copy(z_hbm.at[c].at[pl.ds(s * RPT + k * 128, 128)], zbuf)
        for g in range(8):
            pltpu.async_copy(zbuf.at[pl.ds(g * 16, 16)],
                             pacc.at[bvec(k, g)], psem.at[0], add=True)
            pltpu.async_copy(ones_v, cacc.at[bvec(k, g)], psem.at[1],
                             add=True)
        for g in range(8):
            pltpu.make_async_copy(zbuf.at[pl.ds(g * 16, 16)],
                                  pacc.at[bvec(k, g)], psem.at[0]).wait()
            pltpu.make_async_copy(ones_v, cacc.at[bvec(k, g)],
                                  psem.at[1]).wait()

    plsc.subcore_barrier()
    pltpu.sync_copy(pacc.at[pl.ds(s * rows_b, rows_b)],
                    sums_hbm.at[c].at[pl.ds(s * rows_b, rows_b)])
    pltpu.sync_copy(cacc.at[pl.ds(s * rows_b, rows_b)],
                    cnts_hbm.at[c].at[pl.ds(s * rows_b, rows_b)])


@functools.cache
def _sc_pool_kernel():
    return pl.kernel(
        _sc_pool_body,
        compiler_params=_SC_PARAMS,
        out_type=[jax.ShapeDtypeStruct((NC, B, D), _F32),
                  jax.ShapeDtypeStruct((NC, B, 16), _F32)],
        mesh=_mesh(),
        scratch_types=[
            pltpu.VMEM((8, 128), jnp.int32),
            pltpu.VMEM((128, D), _F32),
            pltpu.VMEM((16, 16), _F32),
            pltpu.VMEM((B // NS, 16), _F32),
            pltpu.VMEM_SHARED((BP, D), _F32),
            pltpu.VMEM_SHARED((BP, 16), _F32),
            pltpu.SemaphoreType.DMA((2,)),
        ],
    )


def _sc_pool(z3, batch2):
    return _sc_pool_kernel()(z3, batch2)


_PREC = jax.lax.Precision.HIGHEST
_ROWS = 1024


def _dot(a, b):
    return jax.lax.dot(a, b, precision=_PREC, preferred_element_type=_F32)


def _tc_pre_body(x_ref, deg_ref, w_ref, h_ref, hp_ref):
    dinv = jax.lax.rsqrt(deg_ref[:, 0:1] + 1.0)
    h = _dot(x_ref[...], w_ref[...])
    h_ref[...] = h
    hp_ref[...] = h * dinv


def _tc_mid_body(acc_ref, h_ref, deg_ref, b_ref, w_ref, f_ref, hn_ref,
                 hpn_ref):
    dinv = jax.lax.rsqrt(deg_ref[:, 0:1] + 1.0)
    zp = dinv * acc_ref[...] + (dinv * dinv) * h_ref[...] + b_ref[...]
    z = jnp.maximum(zp, f_ref[...] * zp)
    hn = _dot(z, w_ref[...])
    hn_ref[...] = hn
    hpn_ref[...] = hn * dinv


def _row_spec():
    return pl.BlockSpec((_ROWS, D), lambda i: (i, 0))


def _deg_spec():
    return pl.BlockSpec((_ROWS, 16), lambda i: (i, 0))


def _bcast_spec(shape):
    return pl.BlockSpec(shape, lambda i: tuple(0 for _ in shape))


def _tc_pre(x2, deg2, W):
    grid = (NC * NP // _ROWS,)
    return pl.pallas_call(
        _tc_pre_body,
        grid=grid,
        in_specs=[_row_spec(), _deg_spec(), _bcast_spec((D, D))],
        out_specs=[_row_spec(), _row_spec()],
        out_shape=[jax.ShapeDtypeStruct((NC * NP, D), _F32)] * 2,
    )(x2, deg2, W)


def _tc_mid(acc, h, deg2, b, Wn, f):
    grid = (NC * NP // _ROWS,)
    return pl.pallas_call(
        _tc_mid_body,
        grid=grid,
        in_specs=[_row_spec(), _row_spec(), _deg_spec(),
                  _bcast_spec((1, D)), _bcast_spec((D, D)),
                  _bcast_spec((1, 1))],
        out_specs=[_row_spec(), _row_spec()],
        out_shape=[jax.ShapeDtypeStruct((NC * NP, D), _F32)] * 2,
    )(acc, h, deg2, b, Wn, f)


def _tc_final_body(sums_ref, cnts_ref, p1a_ref, p1b_ref, b1_ref,
                   p2_ref, b2_ref, o_ref):
    cnt = jnp.maximum(cnts_ref[:, 0:1], 1.0)
    means = sums_ref[...] / cnt
    mp = means[:B]
    mn = means[B:]
    z = jnp.maximum(_dot(mp, p1a_ref[...]) + _dot(mn, p1b_ref[...])
                    + b1_ref[...], 0.0)
    o_ref[...] = _dot(z, p2_ref[...]) + b2_ref[...]


def _tc_final(sums, cnts, P1W, P1b, P2W, P2b):
    p2 = jnp.zeros((D, D), _F32).at[:, :2].set(P2W)
    b2 = jnp.zeros((1, D), _F32).at[:, :2].set(P2b)
    out = pl.pallas_call(
        _tc_final_body,
        out_shape=jax.ShapeDtypeStruct((B, D), _F32),
    )(sums.reshape(NC * B, D), cnts.reshape(NC * B, 16),
      P1W[:D], P1W[D:], P1b.reshape(1, D), p2, b2)
    return out[:, :2]


def _edge_blocks(edge_index, ew):
    pad = EPAD - E
    src = jnp.concatenate([edge_index[0], jnp.zeros((pad,), jnp.int32)])
    dst = jnp.concatenate([edge_index[1], jnp.zeros((pad,), jnp.int32)])
    w = jnp.concatenate([ew, jnp.zeros((pad,), _F32)])
    src = src.reshape(NS, NU, 1, 128)
    dst = dst.reshape(NS, NU, 1, 128)
    w = w.reshape(NS, NU, 1, 128)
    zpad = jnp.zeros((NS, NU, 6, 128), jnp.int32)
    idx = jnp.concatenate([src, dst, zpad], axis=2)
    wblk = jnp.concatenate([w, zpad.astype(_F32),
                            zpad.astype(_F32)[:, :, :1]], axis=2)
    return idx, wblk


def _pad_nodes(x):
    return jnp.pad(x, ((0, NP - N), (0, 0)))


def _pad_batch(batch):
    pad = jnp.full((NP - N,), B, jnp.int32)
    full = jnp.concatenate([batch, pad]).reshape(NS, 5, 128)
    zpad = jnp.zeros((NS, 3, 128), jnp.int32)
    return jnp.concatenate([full, zpad], axis=1)


def kernel(x_pos, edge_index_pos, edge_attr_pos, batch_pos, x_neg, edge_index_neg, edge_attr_neg, batch_neg, W1, b1, W2, b2, W3, b3, P1W, P1b, P2W, P2b):
    eip, ewp = _edge_blocks(edge_index_pos, edge_attr_pos)
    ein, ewn = _edge_blocks(edge_index_neg, edge_attr_neg)
    eidx = jnp.stack([eip, ein])
    ew = jnp.stack([ewp, ewn])
    dip, dwp = _edge_blocks4(edge_index_pos, edge_attr_pos)
    din, dwn = _edge_blocks4(edge_index_neg, edge_attr_neg)
    didx = jnp.stack([dip, din])
    dw = jnp.stack([dwp, dwn])
    x2 = jnp.concatenate([_pad_nodes(x_pos), _pad_nodes(x_neg)], axis=0)
    batch2 = jnp.stack([_pad_batch(batch_pos), _pad_batch(batch_neg)])

    deg = _sc_deg(didx, dw).reshape(NC * NP, 16)

    h1, hp1 = _tc_pre(x2, deg, W1)

    bs = jnp.stack([b1, b2, b3]).reshape(3, 1, D)
    Ws = jnp.stack([W2, W3, jnp.eye(D, dtype=_F32)])
    fs = jnp.array([0.0, 0.0, 1.0], _F32).reshape(3, 1, 1)

    def _step(carry, xs):
        h, hp = carry
        bk, Wk, fk = xs
        acc = _sc_scatter(hp.reshape(NC, NP, D), eidx, ew)
        hn, hpn = _tc_mid(acc.reshape(NC * NP, D), h, deg, bk, Wk, fk)
        return (hn, hpn), None

    (z3, _), _ = lax.scan(_step, (h1, hp1), (bs, Ws, fs))

    sums, cnts = _sc_pool(z3.reshape(NC, NP, D), batch2)
    return _tc_final(sums, cnts, P1W, P1b, P2W, P2b)


# speedup vs baseline: 7.0934x; 1.0610x over previous
import functools

import jax
import jax.numpy as jnp
from jax import lax
from jax.experimental import pallas as pl
from jax.experimental.pallas import tpu as pltpu
from jax.experimental.pallas import tpu_sc as plsc

N = 10000
E = 320000
D = 128
B = 256
NP = 10240
BP = 264

NC = 2
NS = 16
CH = 64
GRP = 4
NG = 80
EPT_CH = NG * GRP
EPAD = NS * EPT_CH * CH
RPT = NP // NS
NB = 4

_F32 = jnp.float32
_SC_PARAMS = pltpu.CompilerParams(needs_layout_passes=False)


@functools.cache
def _mesh():
    return plsc.VectorSubcoreMesh(core_axis_name="c", subcore_axis_name="s",
                                  num_cores=NC, num_subcores=NS)


def _zero_rows(buf, nrows, ncols=D):
    @pl.loop(0, nrows)
    def _(r):
        for j in range(ncols // 16):
            buf[r, pl.ds(j * 16, 16)] = jnp.zeros((16,), _F32)


def _zero_acc_slice(zsrc, acc, s):
    nfull = RPT // CH
    for k in range(nfull):
        pltpu.sync_copy(zsrc, acc.at[pl.ds(s * RPT + k * CH, CH)])
    rem = RPT - nfull * CH
    if rem:
        pltpu.sync_copy(zsrc.at[pl.ds(0, rem)],
                        acc.at[pl.ds(s * RPT + nfull * CH, rem)])


def _sc_deg_body(eb_hbm, ew_hbm, deg_hbm, eb_v, w_v, stage, zc, acc,
                 dsem, esem):
    c = lax.axis_index("c")
    s = lax.axis_index("s")
    eb_g = eb_hbm.at[c].at[s]
    ew_g = ew_hbm.at[c].at[s]
    @pl.loop(0, CH)
    def _(r):
        zc[r, :] = jnp.zeros((16,), _F32)
    nfull = RPT // CH
    for k in range(nfull):
        pltpu.sync_copy(zc, acc.at[pl.ds(s * RPT + k * CH, CH)])
    plsc.subcore_barrier()

    def issue_edges(gi, pb):
        pltpu.async_copy(eb_g.at[gi], eb_v.at[pb], esem.at[pb])
        pltpu.async_copy(ew_g.at[gi], w_v.at[pb], esem.at[pb])

    def wait_edges(pb):
        pltpu.make_async_copy(eb_g.at[0], eb_v.at[pb], esem.at[pb]).wait()
        pltpu.make_async_copy(ew_g.at[0], w_v.at[pb], esem.at[pb]).wait()

    def dvec(pb, k, g):
        off = 256 + k * CH + g * 16
        return eb_v[pb, off // 128, pl.ds(off % 128, 16)]

    def do_group(gi, pb):
        wait_edges(pb)
        @pl.when(gi + 1 < NG)
        def _():
            issue_edges(gi + 1, 1 - pb)
        for k in range(GRP):
            b2 = k % 2
            if k < 2:
                @pl.when(gi > 0)
                def _():
                    for g in range(CH // 16):
                        pltpu.make_async_copy(
                            stage.at[b2].at[pl.ds(g * 16, 16)],
                            acc.at[dvec(1 - pb, k + 2, g)],
                            dsem.at[b2]).wait()
            else:
                for g in range(CH // 16):
                    pltpu.make_async_copy(stage.at[b2].at[pl.ds(g * 16, 16)],
                                          acc.at[dvec(pb, k - 2, g)],
                                          dsem.at[b2]).wait()
            for g in range(CH // 16):
                wrow = w_v[pb, (k * CH + g * 16) // 128,
                           pl.ds((k * CH + g * 16) % 128, 16)]
                for t in range(16):
                    stage[b2, g * 16 + t, :] = jnp.full((16,), wrow[t])
            for g in range(CH // 16):
                pltpu.async_copy(stage.at[b2].at[pl.ds(g * 16, 16)],
                                 acc.at[dvec(pb, k, g)], dsem.at[b2],
                                 add=True)

    issue_edges(0, 0)
    @pl.loop(0, NG, step=2)
    def _(gi):
        do_group(gi, 0)
        do_group(gi + 1, 1)

    for b2 in range(2):
        for g in range(CH // 16):
            pltpu.make_async_copy(stage.at[b2].at[pl.ds(g * 16, 16)],
                                  acc.at[dvec(1, 2 + b2, g)],
                                  dsem.at[b2]).wait()

    plsc.subcore_barrier()
    pltpu.sync_copy(acc.at[pl.ds(s * RPT, RPT)],
                    deg_hbm.at[c].at[pl.ds(s * RPT, RPT)])


@functools.cache
def _sc_deg_kernel():
    return pl.kernel(
        _sc_deg_body,
        compiler_params=_SC_PARAMS,
        out_type=jax.ShapeDtypeStruct((NC, NP, 16), _F32),
        mesh=_mesh(),
        scratch_types=[
            pltpu.VMEM((2, 8, 128), jnp.int32),
            pltpu.VMEM((2, 8, 128), _F32),
            pltpu.VMEM((2, CH, 16), _F32),
            pltpu.VMEM((CH, 16), _F32),
            pltpu.VMEM_SHARED((NP, 16), _F32),
            pltpu.SemaphoreType.DMA((2,)),
            pltpu.SemaphoreType.DMA((2,)),
        ],
    )


def _sc_deg(eidx, ew):
    return _sc_deg_kernel()(eidx, ew)


def _sc_scatter_body(hp_hbm, eb_hbm, ew_hbm, out_hbm, eb_v, w_v, gbuf, acc,
                     gsem, ssem, esem):
    c = lax.axis_index("c")
    s = lax.axis_index("s")
    hp_g = hp_hbm.at[c]
    eb_g = eb_hbm.at[c].at[s]
    ew_g = ew_hbm.at[c].at[s]

    _zero_rows(gbuf.at[0], CH)
    nfull = RPT // CH
    for k in range(nfull):
        pltpu.sync_copy(gbuf.at[0], acc.at[pl.ds(s * RPT + k * CH, CH)])
    plsc.subcore_barrier()

    def issue_edges(gi, pb):
        pltpu.async_copy(eb_g.at[gi], eb_v.at[pb], esem.at[pb])
        pltpu.async_copy(ew_g.at[gi], w_v.at[pb], esem.at[pb])

    def wait_edges(pb):
        pltpu.make_async_copy(eb_g.at[0], eb_v.at[pb], esem.at[pb]).wait()
        pltpu.make_async_copy(ew_g.at[0], w_v.at[pb], esem.at[pb]).wait()

    def svec(pb, k, g):
        off = k * CH + g * 16
        return eb_v[pb, off // 128, pl.ds(off % 128, 16)]

    def dvec(pb, k, g):
        off = 256 + k * CH + g * 16
        return eb_v[pb, off // 128, pl.ds(off % 128, 16)]

    def do_group(gi, pb):
        @pl.when(gi > 0)
        def _():
            for k in range(GRP):
                for g in range(CH // 16):
                    pltpu.make_async_copy(
                        gbuf.at[k].at[pl.ds(g * 16, 16)],
                        acc.at[dvec(1 - pb, k, g)], ssem.at[k]).wait()
        @pl.when(gi + 1 < NG)
        def _():
            issue_edges(gi + 1, 1 - pb)
        for k in range(GRP):
            for g in range(CH // 16):
                pltpu.async_copy(hp_g.at[svec(pb, k, g)],
                                 gbuf.at[k].at[pl.ds(g * 16, 16)],
                                 gsem.at[k])
        for k in range(GRP):
            for g in range(CH // 16):
                pltpu.make_async_copy(hp_g.at[svec(pb, k, g)],
                                      gbuf.at[k].at[pl.ds(g * 16, 16)],
                                      gsem.at[k]).wait()
            for g in range(CH // 16):
                woff = k * CH + g * 16
                wrow = w_v[pb, woff // 128, pl.ds(woff % 128, 16)]
                for t in range(16):
                    e = g * 16 + t
                    wv = jnp.full((16,), wrow[t])
                    for j in range(D // 16):
                        sl = pl.ds(j * 16, 16)
                        gbuf[k, e, sl] = gbuf[k, e, sl] * wv
            for g in range(CH // 16):
                pltpu.async_copy(gbuf.at[k].at[pl.ds(g * 16, 16)],
                                 acc.at[dvec(pb, k, g)], ssem.at[k],
                                 add=True)

    issue_edges(0, 0)
    wait_edges(0)
    @pl.loop(0, NG, step=2)
    def _(gi):
        do_group(gi, 0)
        wait_edges(1)
        do_group(gi + 1, 1)
        @pl.when(gi + 2 < NG)
        def _():
            wait_edges(0)

    for k in range(GRP):
        for g in range(CH // 16):
            pltpu.make_async_copy(gbuf.at[k].at[pl.ds(g * 16, 16)],
                                  acc.at[dvec(1, k, g)], ssem.at[k]).wait()
    plsc.subcore_barrier()

    for k in range(RPT // CH):
        row = s * RPT + k * CH
        pltpu.sync_copy(acc.at[pl.ds(row, CH)],
                        out_hbm.at[c].at[pl.ds(row, CH)])


@functools.cache
def _sc_scatter_kernel():
    return pl.kernel(
        _sc_scatter_body,
        compiler_params=_SC_PARAMS,
        out_type=jax.ShapeDtypeStruct((NC, NP, D), _F32),
        mesh=_mesh(),
        scratch_types=[
            pltpu.VMEM((2, 8, 128), jnp.int32),
            pltpu.VMEM((2, 8, 128), _F32),
            pltpu.VMEM((NB, CH, D), _F32),
            pltpu.VMEM_SHARED((NP, D), _F32),
            pltpu.SemaphoreType.DMA((NB,)),
            pltpu.SemaphoreType.DMA((NB,)),
            pltpu.SemaphoreType.DMA((2,)),
        ],
    )


def _sc_scatter(hp, eidx, ew):
    return _sc_scatter_kernel()(hp, eidx, ew)


def _sc_pool_body(z_hbm, batch_hbm, sums_hbm, cnts_hbm,
                  b_v, zbuf, ones_v, zc, pacc, cacc, psem):
    c = lax.axis_index("c")
    s = lax.axis_index("s")
    rows_b = B // NS
    pltpu.sync_copy(batch_hbm.at[c].at[s], b_v)
    @pl.loop(0, 16)
    def _(r):
        ones_v[r, :] = jnp.full((16,), 1.0)
    @pl.loop(0, rows_b)
    def _(r):
        zc[r, :] = jnp.zeros((16,), _F32)
    _zero_rows(zbuf, rows_b)
    pltpu.sync_copy(zbuf.at[pl.ds(0, rows_b)],
                    pacc.at[pl.ds(s * rows_b, rows_b)])
    pltpu.sync_copy(zc, cacc.at[pl.ds(s * rows_b, rows_b)])
    plsc.subcore_barrier()

    def bvec(k, g):
        off = k * 128 + g * 16
        return b_v[off // 128, pl.ds(off % 128, 16)]

    for k in range(RPT // 128):
        pltpu.sync_copy(z_hbm.at[c].at[pl.ds(s * RPT + k * 128, 128)], zbuf)
        for g in range(8):
            pltpu.async_copy(zbuf.at[pl.ds(g * 16, 16)],
                             pacc.at[bvec(k, g)], psem.at[0], add=True)
            pltpu.async_copy(ones_v, cacc.at[bvec(k, g)], psem.at[1],
                             add=True)
        for g in range(8):
            pltpu.make_async_copy(zbuf.at[pl.ds(g * 16, 16)],
                                  pacc.at[bvec(k, g)], psem.at[0]).wait()
            pltpu.make_async_copy(ones_v, cacc.at[bvec(k, g)],
                                  psem.at[1]).wait()

    plsc.subcore_barrier()
    pltpu.sync_copy(pacc.at[pl.ds(s * rows_b, rows_b)],
                    sums_hbm.at[c].at[pl.ds(s * rows_b, rows_b)])
    pltpu.sync_copy(cacc.at[pl.ds(s * rows_b, rows_b)],
                    cnts_hbm.at[c].at[pl.ds(s * rows_b, rows_b)])


@functools.cache
def _sc_pool_kernel():
    return pl.kernel(
        _sc_pool_body,
        compiler_params=_SC_PARAMS,
        out_type=[jax.ShapeDtypeStruct((NC, B, D), _F32),
                  jax.ShapeDtypeStruct((NC, B, 16), _F32)],
        mesh=_mesh(),
        scratch_types=[
            pltpu.VMEM((8, 128), jnp.int32),
            pltpu.VMEM((128, D), _F32),
            pltpu.VMEM((16, 16), _F32),
            pltpu.VMEM((B // NS, 16), _F32),
            pltpu.VMEM_SHARED((BP, D), _F32),
            pltpu.VMEM_SHARED((BP, 16), _F32),
            pltpu.SemaphoreType.DMA((2,)),
        ],
    )


def _sc_pool(z3, batch2):
    return _sc_pool_kernel()(z3, batch2)


_PREC = jax.lax.Precision.HIGHEST
_ROWS = 1024


def _dot(a, b):
    return jax.lax.dot(a, b, precision=_PREC, preferred_element_type=_F32)


def _tc_pre_body(x_ref, deg_ref, w_ref, h_ref, hp_ref):
    dinv = jax.lax.rsqrt(deg_ref[:, 0:1] + 1.0)
    h = _dot(x_ref[...], w_ref[...])
    h_ref[...] = h
    hp_ref[...] = h * dinv


def _tc_mid_body(acc_ref, h_ref, deg_ref, b_ref, w_ref, f_ref, hn_ref,
                 hpn_ref):
    dinv = jax.lax.rsqrt(deg_ref[:, 0:1] + 1.0)
    zp = dinv * acc_ref[...] + (dinv * dinv) * h_ref[...] + b_ref[...]
    z = jnp.maximum(zp, f_ref[...] * zp)
    hn = _dot(z, w_ref[...])
    hn_ref[...] = hn
    hpn_ref[...] = hn * dinv


def _row_spec():
    return pl.BlockSpec((_ROWS, D), lambda i: (i, 0))


def _deg_spec():
    return pl.BlockSpec((_ROWS, 16), lambda i: (i, 0))


def _bcast_spec(shape):
    return pl.BlockSpec(shape, lambda i: tuple(0 for _ in shape))


def _tc_pre(x2, deg2, W):
    grid = (NC * NP // _ROWS,)
    return pl.pallas_call(
        _tc_pre_body,
        grid=grid,
        in_specs=[_row_spec(), _deg_spec(), _bcast_spec((D, D))],
        out_specs=[_row_spec(), _row_spec()],
        out_shape=[jax.ShapeDtypeStruct((NC * NP, D), _F32)] * 2,
    )(x2, deg2, W)


def _tc_mid(acc, h, deg2, b, Wn, f):
    grid = (NC * NP // _ROWS,)
    return pl.pallas_call(
        _tc_mid_body,
        grid=grid,
        in_specs=[_row_spec(), _row_spec(), _deg_spec(),
                  _bcast_spec((1, D)), _bcast_spec((D, D)),
                  _bcast_spec((1, 1))],
        out_specs=[_row_spec(), _row_spec()],
        out_shape=[jax.ShapeDtypeStruct((NC * NP, D), _F32)] * 2,
    )(acc, h, deg2, b, Wn, f)


def _tc_final_body(sums_ref, cnts_ref, p1a_ref, p1b_ref, b1_ref,
                   p2_ref, b2_ref, o_ref):
    cnt = jnp.maximum(cnts_ref[:, 0:1], 1.0)
    means = sums_ref[...] / cnt
    mp = means[:B]
    mn = means[B:]
    z = jnp.maximum(_dot(mp, p1a_ref[...]) + _dot(mn, p1b_ref[...])
                    + b1_ref[...], 0.0)
    o_ref[...] = _dot(z, p2_ref[...]) + b2_ref[...]


def _tc_final(sums, cnts, P1W, P1b, P2W, P2b):
    p2 = jnp.zeros((D, D), _F32).at[:, :2].set(P2W)
    b2 = jnp.zeros((1, D), _F32).at[:, :2].set(P2b)
    out = pl.pallas_call(
        _tc_final_body,
        out_shape=jax.ShapeDtypeStruct((B, D), _F32),
    )(sums.reshape(NC * B, D), cnts.reshape(NC * B, 16),
      P1W[:D], P1W[D:], P1b.reshape(1, D), p2, b2)
    return out[:, :2]


def _edge_blocks(edge_index, ew):
    pad = EPAD - E
    src = jnp.concatenate([edge_index[0], jnp.zeros((pad,), jnp.int32)])
    dst = jnp.concatenate([edge_index[1], jnp.zeros((pad,), jnp.int32)])
    w = jnp.concatenate([ew, jnp.zeros((pad,), _F32)])
    src = src.reshape(NS, NG, 2, 128)
    dst = dst.reshape(NS, NG, 2, 128)
    w = w.reshape(NS, NG, 2, 128)
    zpad = jnp.zeros((NS, NG, 4, 128), jnp.int32)
    idx = jnp.concatenate([src, dst, zpad], axis=2)
    wblk = jnp.concatenate([w, zpad.astype(_F32),
                            zpad.astype(_F32)[:, :, :2]], axis=2)
    return idx, wblk


def _pad_nodes(x):
    return jnp.pad(x, ((0, NP - N), (0, 0)))


def _pad_batch(batch):
    pad = jnp.full((NP - N,), B, jnp.int32)
    full = jnp.concatenate([batch, pad]).reshape(NS, 5, 128)
    zpad = jnp.zeros((NS, 3, 128), jnp.int32)
    return jnp.concatenate([full, zpad], axis=1)


def kernel(x_pos, edge_index_pos, edge_attr_pos, batch_pos, x_neg, edge_index_neg, edge_attr_neg, batch_neg, W1, b1, W2, b2, W3, b3, P1W, P1b, P2W, P2b):
    eip, ewp = _edge_blocks(edge_index_pos, edge_attr_pos)
    ein, ewn = _edge_blocks(edge_index_neg, edge_attr_neg)
    eidx = jnp.stack([eip, ein])
    ew = jnp.stack([ewp, ewn])
    x2 = jnp.concatenate([_pad_nodes(x_pos), _pad_nodes(x_neg)], axis=0)
    batch2 = jnp.stack([_pad_batch(batch_pos), _pad_batch(batch_neg)])

    deg = _sc_deg(eidx, ew).reshape(NC * NP, 16)

    h1, hp1 = _tc_pre(x2, deg, W1)

    bs = jnp.stack([b1, b2, b3]).reshape(3, 1, D)
    Ws = jnp.stack([W2, W3, jnp.eye(D, dtype=_F32)])
    fs = jnp.array([0.0, 0.0, 1.0], _F32).reshape(3, 1, 1)

    def _step(carry, xs):
        h, hp = carry
        bk, Wk, fk = xs
        acc = _sc_scatter(hp.reshape(NC, NP, D), eidx, ew)
        hn, hpn = _tc_mid(acc.reshape(NC * NP, D), h, deg, bk, Wk, fk)
        return (hn, hpn), None

    (z3, _), _ = lax.scan(_step, (h1, hp1), (bs, Ws, fs))

    sums, cnts = _sc_pool(z3.reshape(NC, NP, D), batch2)
    return _tc_final(sums, cnts, P1W, P1b, P2W, P2b)
